# Initial kernel scaffold; baseline (speedup 1.0000x reference)
import jax
import jax.numpy as jnp
from jax.experimental import pallas as pl


def kernel(inp, table):
    raise NotImplementedError("write your pallas kernel here")


# speedup vs baseline: 1.7970x; 1.7970x over previous
import functools

import jax
import jax.numpy as jnp
from jax import lax
from jax.experimental import pallas as pl
from jax.experimental.pallas import tpu as pltpu
from jax.experimental.pallas import tpu_sc as plsc

BATCH = 16384
HIST = 50
DIM = 64

B_TOTAL = BATCH * HIST
NC = 2
NS = 16
NW = NC * NS
BPW = B_TOTAL // NW
CH = 128
GPC = 4
GROUP = CH * GPC
NGROUPS = BPW // GROUP
IDX_ROWS_PER_W = BPW // CH


def _gather_kernel(idx_hbm, table_hbm, out_hbm, idx_v, rows_v, sem):
    wid = lax.axis_index("s") * NC + lax.axis_index("c")
    base = wid * BPW
    idx_row0 = wid * IDX_ROWS_PER_W

    def body(g, carry):
        pltpu.sync_copy(idx_hbm.at[pl.ds(idx_row0 + g * GPC, GPC)], idx_v)
        copies = [
            pltpu.async_copy(
                table_hbm.at[idx_v.at[j]],
                rows_v.at[pl.ds(j * CH, CH)],
                sem,
            )
            for j in range(GPC)
        ]
        for cp in copies:
            cp.wait()
        pltpu.sync_copy(rows_v, out_hbm.at[pl.ds(base + g * GROUP, GROUP)])
        return carry

    lax.fori_loop(0, NGROUPS, body, 0)


def kernel(inp, table):
    idx2d = inp.reshape(B_TOTAL // CH, CH).astype(jnp.int32)
    mesh = plsc.VectorSubcoreMesh(core_axis_name="c", subcore_axis_name="s")
    run = functools.partial(
        pl.kernel,
        mesh=mesh,
        out_type=jax.ShapeDtypeStruct((B_TOTAL, DIM), jnp.float32),
        scratch_types=[
            pltpu.VMEM((GPC, CH), jnp.int32),
            pltpu.VMEM((GROUP, DIM), jnp.float32),
            pltpu.SemaphoreType.DMA,
        ],
        compiler_params=pltpu.CompilerParams(use_tc_tiling_on_sc=False),
    )(_gather_kernel)
    out = run(idx2d, table)
    return out.reshape(BATCH, HIST, DIM)


# speedup vs baseline: 1.8702x; 1.0408x over previous
import functools

import jax
import jax.numpy as jnp
from jax import lax
from jax.experimental import pallas as pl
from jax.experimental.pallas import tpu as pltpu
from jax.experimental.pallas import tpu_sc as plsc

BATCH = 16384
HIST = 50
DIM = 64

B_TOTAL = BATCH * HIST
NC = 2
NS = 16
NW = NC * NS
BPW = B_TOTAL // NW
CH = 128
GPC = 4
GROUP = CH * GPC
NGROUPS = BPW // GROUP
NPAIRS = NGROUPS // 2
IDX_ROWS_PER_W = BPW // CH


def _gather_kernel(idx_hbm, table_hbm, out_hbm, idx_all, rows_v, gsem, ssem):
    wid = lax.axis_index("s") * NC + lax.axis_index("c")
    base = wid * BPW
    pltpu.sync_copy(idx_hbm.at[pl.ds(wid * IDX_ROWS_PER_W, IDX_ROWS_PER_W)], idx_all)

    def fire_gathers(g, b):
        for j in range(GPC):
            pltpu.async_copy(
                table_hbm.at[idx_all.at[g * GPC + j]],
                rows_v.at[b, pl.ds(j * CH, CH)],
                gsem,
            )

    def wait_gathers(b):
        for j in range(GPC):
            pltpu.make_async_copy(
                table_hbm.at[idx_all.at[j]],
                rows_v.at[b, pl.ds(j * CH, CH)],
                gsem,
            ).wait()

    def start_store(g, b):
        pltpu.async_copy(rows_v.at[b], out_hbm.at[pl.ds(base + g * GROUP, GROUP)], ssem)

    def wait_store():
        pltpu.make_async_copy(
            rows_v.at[0], out_hbm.at[pl.ds(base, GROUP)], ssem
        ).wait()

    fire_gathers(0, 0)

    def body(p, carry):
        g0 = p * 2
        wait_gathers(0)
        start_store(g0, 0)

        @pl.when(p > 0)
        def _():
            wait_store()

        fire_gathers(g0 + 1, 1)
        wait_gathers(1)
        start_store(g0 + 1, 1)
        wait_store()

        @pl.when(p < NPAIRS - 1)
        def _():
            fire_gathers(g0 + 2, 0)

        return carry

    lax.fori_loop(0, NPAIRS, body, 0)
    wait_store()


def kernel(inp, table):
    idx2d = inp.reshape(B_TOTAL // CH, CH).astype(jnp.int32)
    mesh = plsc.VectorSubcoreMesh(core_axis_name="c", subcore_axis_name="s")
    run = functools.partial(
        pl.kernel,
        mesh=mesh,
        out_type=jax.ShapeDtypeStruct((B_TOTAL, DIM), jnp.float32),
        scratch_types=[
            pltpu.VMEM((IDX_ROWS_PER_W, CH), jnp.int32),
            pltpu.VMEM((2, GROUP, DIM), jnp.float32),
            pltpu.SemaphoreType.DMA,
            pltpu.SemaphoreType.DMA,
        ],
        compiler_params=pltpu.CompilerParams(use_tc_tiling_on_sc=False),
    )(_gather_kernel)
    out = run(idx2d, table)
    return out.reshape(BATCH, HIST, DIM)
